# baseline (speedup 1.0000x reference)
import functools

import jax
import jax.numpy as jnp
from jax import lax
from jax.experimental import pallas as pl
from jax.experimental.pallas import tpu as pltpu
from jax.experimental.pallas import tpu_sc as plsc

VOCAB = 1000
EMBED = 128
B = 4096
S = 50

NW = 32
CAP_W = B // NW
G = 2
NCH = CAP_W // G
NBUF = 4
NGRP = NCH // NBUF

_mesh = plsc.VectorSubcoreMesh(core_axis_name="c", subcore_axis_name="s")


@functools.partial(
    pl.kernel,
    mesh=_mesh,
    out_type=jax.ShapeDtypeStruct((B, S, EMBED), jnp.float32),
    scratch_types=[
        pltpu.VMEM((CAP_W, S), jnp.int32),
        pltpu.VMEM((NBUF, G, S, EMBED), jnp.float32),
        pltpu.VMEM_SHARED((VOCAB, EMBED), jnp.float32),
    ] + [pltpu.SemaphoreType.DMA] * (2 * NBUF),
    compiler_params=pltpu.CompilerParams(use_tc_tiling_on_sc=True),
)
def _emb_lookup(table_hbm, idx_hbm, out_hbm, idx_v, rows_v, table_sp, *sems):
    gsems, wsems = sems[:NBUF], sems[NBUF:]
    sid = lax.axis_index("s")
    wid = sid * 2 + lax.axis_index("c")
    base = wid * CAP_W

    @pl.when(sid == 0)
    def _():
        pltpu.sync_copy(table_hbm, table_sp)

    pltpu.sync_copy(idx_hbm.at[pl.ds(base, CAP_W)], idx_v)
    plsc.subcore_barrier()

    def gathers(j, b):
        return [
            pltpu.make_async_copy(
                table_sp.at[idx_v.at[j * G + g]], rows_v.at[b, g], gsems[b])
            for g in range(G)
        ]

    def writeback(j, b):
        return pltpu.make_async_copy(
            rows_v.at[b], out_hbm.at[pl.ds(base + j * G, G)], wsems[b])

    for b in range(NBUF):
        for cp in gathers(b, b):
            cp.start()

    def group(g, carry):
        for b in range(NBUF):
            j = g * NBUF + b
            for cp in gathers(j, b):
                cp.wait()
            writeback(j, b).start()

            @pl.when(g != NGRP - 1)
            def _():
                writeback(j, b).wait()
                for cp in gathers(j + NBUF, b):
                    cp.start()

        return carry

    lax.fori_loop(0, NGRP, group, 0)

    for b in range(NBUF):
        writeback((NGRP - 1) * NBUF + b, b).wait()


def kernel(captions, weights):
    return _emb_lookup(weights, captions)


# speedup vs baseline: 2.1518x; 2.1518x over previous
import functools

import jax
import jax.numpy as jnp
from jax import lax
from jax.experimental import pallas as pl
from jax.experimental.pallas import tpu as pltpu
from jax.experimental.pallas import tpu_sc as plsc

VOCAB = 1000
EMBED = 128
B = 4096
S = 50

NW = 32
BLK = B // NW
NBUF = 5
NGRP = S // NBUF

_mesh = plsc.VectorSubcoreMesh(core_axis_name="c", subcore_axis_name="s")


@functools.partial(
    pl.kernel,
    mesh=_mesh,
    out_type=jax.ShapeDtypeStruct((S, B, EMBED), jnp.float32),
    scratch_types=[
        pltpu.VMEM((S, BLK), jnp.int32),
        pltpu.VMEM((NBUF, BLK, EMBED), jnp.float32),
        pltpu.VMEM_SHARED((VOCAB, EMBED), jnp.float32),
    ] + [pltpu.SemaphoreType.DMA] * (2 * NBUF),
)
def _emb_lookup(table_hbm, idx_hbm, out_hbm, idx_v, rows_v, table_sp, *sems):
    gsems, wsems = sems[:NBUF], sems[NBUF:]
    sid = lax.axis_index("s")
    wid = sid * 2 + lax.axis_index("c")
    base = wid * BLK

    @pl.when(sid == 0)
    def _():
        pltpu.sync_copy(table_hbm, table_sp)

    pltpu.sync_copy(idx_hbm.at[pl.ds(0, S), pl.ds(base, BLK)], idx_v)
    plsc.subcore_barrier()

    def gather(j, b):
        return pltpu.make_async_copy(
            table_sp.at[idx_v.at[j]], rows_v.at[b], gsems[b])

    def writeback(j, b):
        return pltpu.make_async_copy(
            rows_v.at[b], out_hbm.at[j, pl.ds(base, BLK)], wsems[b])

    for b in range(NBUF):
        gather(b, b).start()

    def group(g, carry):
        for b in range(NBUF):
            j = g * NBUF + b
            gather(j, b).wait()
            writeback(j, b).start()

            @pl.when(g != NGRP - 1)
            def _():
                writeback(j, b).wait()
                gather(j + NBUF, b).start()

        return carry

    lax.fori_loop(0, NGRP, group, 0)

    for b in range(NBUF):
        writeback((NGRP - 1) * NBUF + b, b).wait()


def kernel(captions, weights):
    out2 = _emb_lookup(weights, captions.T)
    return out2.transpose(1, 0, 2)
